# baseline (speedup 1.0000x reference)
import jax
import jax.numpy as jnp
from jax.experimental import pallas as pl
from jax.experimental.pallas import tpu as pltpu

_EPS = 1e-5
_CDT = jnp.bfloat16


def _fused_body(x1_c, x1_t, x1_b, x2_c, x2_t2, x2_t1, x2_b1, x2_b2,
                wu0, wu1, bu, wc1, wm1, s1, f1, wc2, wm2, s2, f2, o_ref):
    T, W1, L = o_ref.shape
    Cout = L // 2
    T2 = x1_c.shape[0]
    C1 = x1_c.shape[-1]
    r = pl.program_id(1)
    last = pl.num_programs(1) - 1
    rI = T2 + 2

    x1slab = jnp.concatenate([x1_t[...], x1_c[...], x1_b[...]], axis=0)
    x2d = x1slab.reshape(rI * x1slab.shape[1], C1)
    y0 = jnp.dot(x2d, wu0[...], preferred_element_type=jnp.float32) + bu[...]
    y1 = jnp.dot(x2d, wu1[...], preferred_element_type=jnp.float32) + bu[...]
    Lu = y0.shape[-1]
    u0 = y0.reshape(rI, W1, Lu)
    u1 = y1.reshape(rI, W1, Lu)
    up = jnp.stack([u0, u1], axis=1).reshape(2 * rI, W1, Lu)
    utop = jnp.where(r > 0, up[:2], jnp.zeros_like(up[:2]))
    ubot = jnp.where(r < last, up[-2:], jnp.zeros_like(up[-2:]))
    upslab = jnp.concatenate([utop, up[2:-2], ubot], axis=0).astype(_CDT)

    t2 = jnp.where(r > 0, x2_t2[...], jnp.zeros_like(x2_t2[...]))
    t1 = jnp.where(r > 0, x2_t1[...], jnp.zeros_like(x2_t1[...]))
    b1 = jnp.where(r < last, x2_b1[...], jnp.zeros_like(x2_b1[...]))
    b2 = jnp.where(r < last, x2_b2[...], jnp.zeros_like(x2_b2[...]))
    x2slab = jnp.concatenate([t2, t1, x2_c[...], b1, b2], axis=0)

    def conv3x3(slabs, n_rows, wc, wm, scale, shift):
        pieces = []
        for k in range(3):
            for slab in slabs:
                pieces.append(slab[k:k + n_rows].reshape(n_rows * W1,
                                                         slab.shape[-1]))
        patch = pieces[0] if len(pieces) == 1 else jnp.concatenate(pieces, -1)
        zc = jnp.dot(patch, wc[...], preferred_element_type=jnp.float32)
        zm = jnp.dot(patch, wm[...], preferred_element_type=jnp.float32)
        M = n_rows * W1
        colj = jax.lax.broadcasted_iota(jnp.int32, zc.shape, 0) % W1
        lane = jax.lax.broadcasted_iota(jnp.int32, zc.shape, 1)
        zero = jnp.zeros_like(zc)
        acc = zc
        acc = acc + jnp.where((colj > 0) & (lane < Cout),
                              pltpu.roll(zm, shift=1, axis=0), zero)
        acc = acc + jnp.where((colj < W1 - 1) & (lane >= Cout),
                              pltpu.roll(zm, shift=M - 1, axis=0), zero)
        y = jnp.maximum(acc * scale[...] + shift[...], 0.0)
        return y.reshape(n_rows, W1, 2 * Cout)

    h3 = conv3x3([x2slab, upslab], T + 2, wc1, wm1, s1, f1)
    htop = jnp.where(r > 0, h3[:1], jnp.zeros_like(h3[:1]))
    hbot = jnp.where(r < last, h3[-1:], jnp.zeros_like(h3[-1:]))
    hslab = jnp.concatenate([htop, h3[1:-1], hbot], axis=0).astype(_CDT)

    o_ref[...] = conv3x3([hslab], T, wc2, wm2, s2, f2).astype(o_ref.dtype)


def _packed_conv_weights(w_torch, src_chans):
    Co = w_torch.shape[0]
    wc_parts, wm_parts = [], []
    off = 0
    for C in src_chans:
        w = w_torch[:, off:off + C]
        wt = jnp.transpose(w, (2, 1, 3, 0))
        z = jnp.zeros_like(wt[:, :, 0])
        dj0 = jnp.stack([wt[:, :, 1], wt[:, :, 0]], axis=2)
        dj1 = jnp.stack([wt[:, :, 2], wt[:, :, 1]], axis=2)
        wc = jnp.stack([dj0, dj1], axis=1)
        mdj0 = jnp.stack([z, wt[:, :, 2]], axis=2)
        mdj1 = jnp.stack([wt[:, :, 0], z], axis=2)
        wm = jnp.stack([mdj0, mdj1], axis=1)
        wc_parts.append(wc.reshape(3, 2 * C, 2 * Co))
        wm_parts.append(wm.reshape(3, 2 * C, 2 * Co))
        off += C
    wc = jnp.concatenate(wc_parts, axis=1).reshape(-1, 2 * Co)
    wm = jnp.concatenate(wm_parts, axis=1).reshape(-1, 2 * Co)
    return wc.astype(_CDT), wm.astype(_CDT)


def _dup(v):
    return jnp.concatenate([v, v]).astype(jnp.float32).reshape(1, -1)


def kernel(up_w, up_b, c1_w, c1_b, bn1_g, bn1_b, bn1_m, bn1_v,
           c2_w, c2_b, bn2_g, bn2_b, bn2_m, bn2_v, x1, x2):
    B, C1, H1, W1x = x1.shape
    _, C2, H2, W2 = x2.shape
    Cu = up_w.shape[1]
    Co1 = c1_w.shape[0]
    Co2 = c2_w.shape[0]
    W1 = W2 // 2


    wt = jnp.transpose(up_w, (2, 3, 0, 1))
    wu0 = jnp.transpose(wt[0], (1, 0, 2)).reshape(C1, 2 * Cu).astype(_CDT)
    wu1 = jnp.transpose(wt[1], (1, 0, 2)).reshape(C1, 2 * Cu).astype(_CDT)
    bu = _dup(up_b)

    wc1, wm1 = _packed_conv_weights(c1_w, [C2, Cu])
    wc2, wm2 = _packed_conv_weights(c2_w, [Co1])
    K1, K2 = wc1.shape[0], wc2.shape[0]

    sc1 = bn1_g / jnp.sqrt(bn1_v + _EPS)
    sh1 = _dup((c1_b - bn1_m) * sc1 + bn1_b)
    sc1 = _dup(sc1)
    sc2 = bn2_g / jnp.sqrt(bn2_v + _EPS)
    sh2 = _dup((c2_b - bn2_m) * sc2 + bn2_b)
    sc2 = _dup(sc2)

    T = 64
    T2 = T // 2
    grid = (B, H2 // T)

    in_specs = [
        pl.BlockSpec((None, T2, W1x, C1), lambda bi, r: (bi, r, 0, 0)),
        pl.BlockSpec((None, 1, W1x, C1),
                     lambda bi, r: (bi, jnp.maximum(r * T2 - 1, 0), 0, 0)),
        pl.BlockSpec((None, 1, W1x, C1),
                     lambda bi, r: (bi, jnp.minimum((r + 1) * T2, H1 - 1), 0, 0)),
        pl.BlockSpec((None, T, W1, 2 * C2), lambda bi, r: (bi, r, 0, 0)),
        pl.BlockSpec((None, 1, W1, 2 * C2),
                     lambda bi, r: (bi, jnp.maximum(r * T - 2, 0), 0, 0)),
        pl.BlockSpec((None, 1, W1, 2 * C2),
                     lambda bi, r: (bi, jnp.maximum(r * T - 1, 0), 0, 0)),
        pl.BlockSpec((None, 1, W1, 2 * C2),
                     lambda bi, r: (bi, jnp.minimum((r + 1) * T, H2 - 1), 0, 0)),
        pl.BlockSpec((None, 1, W1, 2 * C2),
                     lambda bi, r: (bi, jnp.minimum((r + 1) * T + 1, H2 - 1), 0, 0)),
        pl.BlockSpec((C1, 2 * Cu), lambda bi, r: (0, 0)),
        pl.BlockSpec((C1, 2 * Cu), lambda bi, r: (0, 0)),
        pl.BlockSpec((1, 2 * Cu), lambda bi, r: (0, 0)),
        pl.BlockSpec((K1, 2 * Co1), lambda bi, r: (0, 0)),
        pl.BlockSpec((K1, 2 * Co1), lambda bi, r: (0, 0)),
        pl.BlockSpec((1, 2 * Co1), lambda bi, r: (0, 0)),
        pl.BlockSpec((1, 2 * Co1), lambda bi, r: (0, 0)),
        pl.BlockSpec((K2, 2 * Co2), lambda bi, r: (0, 0)),
        pl.BlockSpec((K2, 2 * Co2), lambda bi, r: (0, 0)),
        pl.BlockSpec((1, 2 * Co2), lambda bi, r: (0, 0)),
        pl.BlockSpec((1, 2 * Co2), lambda bi, r: (0, 0)),
    ]

    flops = 2 * B * H2 * W2 * (C1 * Cu + 3 * 3 * (C2 + Cu) * Co1
                               + 3 * 3 * Co1 * Co2)
    bytes_acc = (B * H1 * W1x * C1 + B * H2 * W2 * C2 + B * H2 * W2 * Co2) * 2

    def run(x1s, x2s):
        nb = x1s.shape[0]
        x1h = jnp.transpose(x1s, (0, 2, 3, 1)).astype(_CDT)
        x2p = jnp.transpose(x2s, (0, 2, 3, 1)).reshape(
            nb, H2, W1, 2 * C2).astype(_CDT)
        out = pl.pallas_call(
            _fused_body,
            out_shape=jax.ShapeDtypeStruct((nb, H2, W1, 2 * Co2), _CDT),
            grid=(nb, H2 // T),
            in_specs=in_specs,
            out_specs=pl.BlockSpec((None, T, W1, 2 * Co2),
                                   lambda bi, r: (bi, r, 0, 0)),
            compiler_params=pltpu.CompilerParams(
                dimension_semantics=("parallel", "parallel"),
                vmem_limit_bytes=100 * 1024 * 1024),
            cost_estimate=pl.CostEstimate(
                flops=int(flops) // (B // nb), transcendentals=0,
                bytes_accessed=int(bytes_acc) // (B // nb)),
        )(x1h, x1h, x1h, x2p, x2p, x2p, x2p, x2p,
          wu0, wu1, bu, wc1, wm1, sc1, sh1, wc2, wm2, sc2, sh2)
        out = out.reshape(nb, H2, W1, 2, Co2)
        return jnp.transpose(out, (0, 4, 1, 2, 3)).reshape(
            nb, Co2, H2, W2).astype(jnp.float32)

    if B % 2 == 0 and B > 1:
        h = B // 2
        return jnp.concatenate([run(x1[:h], x2[:h]), run(x1[h:], x2[h:])],
                               axis=0)
    return run(x1, x2)


# speedup vs baseline: 1.1506x; 1.1506x over previous
import jax
import jax.numpy as jnp
from jax.experimental import pallas as pl
from jax.experimental.pallas import tpu as pltpu

_EPS = 1e-5
_CDT = jnp.bfloat16


def _fused_body(x1_c, x1_t, x1_b, x2_c, x2_t2, x2_t1, x2_b1, x2_b2,
                wu0, wu1, bu, wc1, wm1, s1, f1, wc2, wm2, s2, f2, o_ref):
    T, W1, L = o_ref.shape
    Cout = L // 2
    T2 = x1_c.shape[0]
    C1 = x1_c.shape[-1]
    r = pl.program_id(1)
    last = pl.num_programs(1) - 1
    rI = T2 + 2

    x1slab = jnp.concatenate([x1_t[...], x1_c[...], x1_b[...]], axis=0)
    x2d = x1slab.reshape(rI * x1slab.shape[1], C1)
    y0 = jnp.dot(x2d, wu0[...], preferred_element_type=jnp.float32) + bu[...]
    y1 = jnp.dot(x2d, wu1[...], preferred_element_type=jnp.float32) + bu[...]
    Lu = y0.shape[-1]
    u0 = y0.reshape(rI, W1, Lu)
    u1 = y1.reshape(rI, W1, Lu)
    up = jnp.stack([u0, u1], axis=1).reshape(2 * rI, W1, Lu)
    utop = jnp.where(r > 0, up[:2], jnp.zeros_like(up[:2]))
    ubot = jnp.where(r < last, up[-2:], jnp.zeros_like(up[-2:]))
    upslab = jnp.concatenate([utop, up[2:-2], ubot], axis=0).astype(_CDT)

    t2 = jnp.where(r > 0, x2_t2[...], jnp.zeros_like(x2_t2[...]))
    t1 = jnp.where(r > 0, x2_t1[...], jnp.zeros_like(x2_t1[...]))
    b1 = jnp.where(r < last, x2_b1[...], jnp.zeros_like(x2_b1[...]))
    b2 = jnp.where(r < last, x2_b2[...], jnp.zeros_like(x2_b2[...]))
    x2slab = jnp.concatenate([t2, t1, x2_c[...], b1, b2], axis=0)

    def conv3x3(slabs, n_rows, wc, wm, scale, shift):
        pieces = []
        for k in range(3):
            for slab in slabs:
                pieces.append(slab[k:k + n_rows].reshape(n_rows * W1,
                                                         slab.shape[-1]))
        patch = pieces[0] if len(pieces) == 1 else jnp.concatenate(pieces, -1)
        zc = jnp.dot(patch, wc[...], preferred_element_type=jnp.float32)
        zm = jnp.dot(patch, wm[...], preferred_element_type=jnp.float32)
        M = n_rows * W1
        colj = jax.lax.broadcasted_iota(jnp.int32, zc.shape, 0) % W1
        lane = jax.lax.broadcasted_iota(jnp.int32, zc.shape, 1)
        zero = jnp.zeros_like(zc)
        acc = zc
        acc = acc + jnp.where((colj > 0) & (lane < Cout),
                              pltpu.roll(zm, shift=1, axis=0), zero)
        acc = acc + jnp.where((colj < W1 - 1) & (lane >= Cout),
                              pltpu.roll(zm, shift=M - 1, axis=0), zero)
        y = jnp.maximum(acc * scale[...] + shift[...], 0.0)
        return y.reshape(n_rows, W1, 2 * Cout)

    h3 = conv3x3([x2slab, upslab], T + 2, wc1, wm1, s1, f1)
    htop = jnp.where(r > 0, h3[:1], jnp.zeros_like(h3[:1]))
    hbot = jnp.where(r < last, h3[-1:], jnp.zeros_like(h3[-1:]))
    hslab = jnp.concatenate([htop, h3[1:-1], hbot], axis=0).astype(_CDT)

    o_ref[...] = conv3x3([hslab], T, wc2, wm2, s2, f2).astype(o_ref.dtype)


def _packed_conv_weights(w_torch, src_chans):
    Co = w_torch.shape[0]
    wc_parts, wm_parts = [], []
    off = 0
    for C in src_chans:
        w = w_torch[:, off:off + C]
        wt = jnp.transpose(w, (2, 1, 3, 0))
        z = jnp.zeros_like(wt[:, :, 0])
        dj0 = jnp.stack([wt[:, :, 1], wt[:, :, 0]], axis=2)
        dj1 = jnp.stack([wt[:, :, 2], wt[:, :, 1]], axis=2)
        wc = jnp.stack([dj0, dj1], axis=1)
        mdj0 = jnp.stack([z, wt[:, :, 2]], axis=2)
        mdj1 = jnp.stack([wt[:, :, 0], z], axis=2)
        wm = jnp.stack([mdj0, mdj1], axis=1)
        wc_parts.append(wc.reshape(3, 2 * C, 2 * Co))
        wm_parts.append(wm.reshape(3, 2 * C, 2 * Co))
        off += C
    wc = jnp.concatenate(wc_parts, axis=1).reshape(-1, 2 * Co)
    wm = jnp.concatenate(wm_parts, axis=1).reshape(-1, 2 * Co)
    return wc.astype(_CDT), wm.astype(_CDT)


def _dup(v):
    return jnp.concatenate([v, v]).astype(jnp.float32).reshape(1, -1)


def kernel(up_w, up_b, c1_w, c1_b, bn1_g, bn1_b, bn1_m, bn1_v,
           c2_w, c2_b, bn2_g, bn2_b, bn2_m, bn2_v, x1, x2):
    B, C1, H1, W1x = x1.shape
    _, C2, H2, W2 = x2.shape
    Cu = up_w.shape[1]
    Co1 = c1_w.shape[0]
    Co2 = c2_w.shape[0]
    W1 = W2 // 2

    x1h = jnp.transpose(x1, (0, 2, 3, 1)).astype(_CDT)
    x2p = jnp.transpose(x2, (0, 2, 3, 1)).reshape(B, H2, W1, 2 * C2).astype(_CDT)

    wt = jnp.transpose(up_w, (2, 3, 0, 1))
    wu0 = jnp.transpose(wt[0], (1, 0, 2)).reshape(C1, 2 * Cu).astype(_CDT)
    wu1 = jnp.transpose(wt[1], (1, 0, 2)).reshape(C1, 2 * Cu).astype(_CDT)
    bu = _dup(up_b)

    wc1, wm1 = _packed_conv_weights(c1_w, [C2, Cu])
    wc2, wm2 = _packed_conv_weights(c2_w, [Co1])
    K1, K2 = wc1.shape[0], wc2.shape[0]

    sc1 = bn1_g / jnp.sqrt(bn1_v + _EPS)
    sh1 = _dup((c1_b - bn1_m) * sc1 + bn1_b)
    sc1 = _dup(sc1)
    sc2 = bn2_g / jnp.sqrt(bn2_v + _EPS)
    sh2 = _dup((c2_b - bn2_m) * sc2 + bn2_b)
    sc2 = _dup(sc2)

    T = min(64, H2)
    T2 = T // 2
    grid = (B, H2 // T)

    in_specs = [
        pl.BlockSpec((None, T2, W1x, C1), lambda bi, r: (bi, r, 0, 0)),
        pl.BlockSpec((None, 1, W1x, C1),
                     lambda bi, r: (bi, jnp.maximum(r * T2 - 1, 0), 0, 0)),
        pl.BlockSpec((None, 1, W1x, C1),
                     lambda bi, r: (bi, jnp.minimum((r + 1) * T2, H1 - 1), 0, 0)),
        pl.BlockSpec((None, T, W1, 2 * C2), lambda bi, r: (bi, r, 0, 0)),
        pl.BlockSpec((None, 1, W1, 2 * C2),
                     lambda bi, r: (bi, jnp.maximum(r * T - 2, 0), 0, 0)),
        pl.BlockSpec((None, 1, W1, 2 * C2),
                     lambda bi, r: (bi, jnp.maximum(r * T - 1, 0), 0, 0)),
        pl.BlockSpec((None, 1, W1, 2 * C2),
                     lambda bi, r: (bi, jnp.minimum((r + 1) * T, H2 - 1), 0, 0)),
        pl.BlockSpec((None, 1, W1, 2 * C2),
                     lambda bi, r: (bi, jnp.minimum((r + 1) * T + 1, H2 - 1), 0, 0)),
        pl.BlockSpec((C1, 2 * Cu), lambda bi, r: (0, 0)),
        pl.BlockSpec((C1, 2 * Cu), lambda bi, r: (0, 0)),
        pl.BlockSpec((1, 2 * Cu), lambda bi, r: (0, 0)),
        pl.BlockSpec((K1, 2 * Co1), lambda bi, r: (0, 0)),
        pl.BlockSpec((K1, 2 * Co1), lambda bi, r: (0, 0)),
        pl.BlockSpec((1, 2 * Co1), lambda bi, r: (0, 0)),
        pl.BlockSpec((1, 2 * Co1), lambda bi, r: (0, 0)),
        pl.BlockSpec((K2, 2 * Co2), lambda bi, r: (0, 0)),
        pl.BlockSpec((K2, 2 * Co2), lambda bi, r: (0, 0)),
        pl.BlockSpec((1, 2 * Co2), lambda bi, r: (0, 0)),
        pl.BlockSpec((1, 2 * Co2), lambda bi, r: (0, 0)),
    ]

    flops = 2 * B * H2 * W2 * (C1 * Cu + 3 * 3 * (C2 + Cu) * Co1
                               + 3 * 3 * Co1 * Co2)
    bytes_acc = (B * H1 * W1x * C1 + B * H2 * W2 * C2 + B * H2 * W2 * Co2) * 2

    out = pl.pallas_call(
        _fused_body,
        out_shape=jax.ShapeDtypeStruct((B, H2, W1, 2 * Co2), _CDT),
        grid=grid,
        in_specs=in_specs,
        out_specs=pl.BlockSpec((None, T, W1, 2 * Co2),
                               lambda bi, r: (bi, r, 0, 0)),
        compiler_params=pltpu.CompilerParams(
            dimension_semantics=("parallel", "parallel"),
            vmem_limit_bytes=100 * 1024 * 1024),
        cost_estimate=pl.CostEstimate(flops=int(flops), transcendentals=0,
                                      bytes_accessed=int(bytes_acc)),
    )(x1h, x1h, x1h, x2p, x2p, x2p, x2p, x2p,
      wu0, wu1, bu, wc1, wm1, sc1, sh1, wc2, wm2, sc2, sh2)

    out = out.reshape(B, H2, W1, 2, Co2)
    return jnp.transpose(out, (0, 4, 1, 2, 3)).reshape(
        B, Co2, H2, W2).astype(jnp.float32)


# speedup vs baseline: 1.1763x; 1.0223x over previous
import jax
import jax.numpy as jnp
from jax.experimental import pallas as pl
from jax.experimental.pallas import tpu as pltpu

_EPS = 1e-5
_CDT = jnp.bfloat16


def _fused_body(x1_c, x1_t, x1_b, x2_c, x2_t2, x2_t1, x2_b1, x2_b2,
                wu0, wu1, bu, wc1, wm1, s1, f1, wc2, wm2, s2, f2, o_ref):
    T, W1, L = o_ref.shape
    Cout = L // 2
    T2 = x1_c.shape[0]
    C1 = x1_c.shape[-1]
    r = pl.program_id(1)
    last = pl.num_programs(1) - 1
    rI = T2 + 2

    x1slab = jnp.concatenate([x1_t[...], x1_c[...], x1_b[...]], axis=0)
    x2d = x1slab.reshape(rI * x1slab.shape[1], C1)
    y0 = jnp.dot(x2d, wu0[...], preferred_element_type=jnp.float32) + bu[...]
    y1 = jnp.dot(x2d, wu1[...], preferred_element_type=jnp.float32) + bu[...]
    Lu = y0.shape[-1]
    u0 = y0.reshape(rI, W1, Lu)
    u1 = y1.reshape(rI, W1, Lu)
    up = jnp.stack([u0, u1], axis=1).reshape(2 * rI, W1, Lu)
    utop = jnp.where(r > 0, up[:2], jnp.zeros_like(up[:2]))
    ubot = jnp.where(r < last, up[-2:], jnp.zeros_like(up[-2:]))
    upslab = jnp.concatenate([utop, up[2:-2], ubot], axis=0).astype(_CDT)

    t2 = jnp.where(r > 0, x2_t2[...], jnp.zeros_like(x2_t2[...]))
    t1 = jnp.where(r > 0, x2_t1[...], jnp.zeros_like(x2_t1[...]))
    b1 = jnp.where(r < last, x2_b1[...], jnp.zeros_like(x2_b1[...]))
    b2 = jnp.where(r < last, x2_b2[...], jnp.zeros_like(x2_b2[...]))
    x2slab = jnp.concatenate([t2, t1, x2_c[...], b1, b2], axis=0)

    def conv3x3(slabs, n_rows, wc, wm, scale, shift):
        pieces = []
        for k in range(3):
            for slab in slabs:
                pieces.append(slab[k:k + n_rows].reshape(n_rows * W1,
                                                         slab.shape[-1]))
        patch = pieces[0] if len(pieces) == 1 else jnp.concatenate(pieces, -1)
        zc = jnp.dot(patch, wc[...], preferred_element_type=jnp.float32)
        zm = jnp.dot(patch, wm[...], preferred_element_type=jnp.float32)
        M = n_rows * W1
        colj = jax.lax.broadcasted_iota(jnp.int32, zc.shape, 0) % W1
        lane = jax.lax.broadcasted_iota(jnp.int32, zc.shape, 1)
        zero = jnp.zeros_like(zc)
        acc = zc
        acc = acc + jnp.where((colj > 0) & (lane < Cout),
                              pltpu.roll(zm, shift=1, axis=0), zero)
        acc = acc + jnp.where((colj < W1 - 1) & (lane >= Cout),
                              pltpu.roll(zm, shift=M - 1, axis=0), zero)
        y = jnp.maximum(acc * scale[...] + shift[...], 0.0)
        return y.reshape(n_rows, W1, 2 * Cout)

    h3 = conv3x3([x2slab, upslab], T + 2, wc1, wm1, s1, f1)
    htop = jnp.where(r > 0, h3[:1], jnp.zeros_like(h3[:1]))
    hbot = jnp.where(r < last, h3[-1:], jnp.zeros_like(h3[-1:]))
    hslab = jnp.concatenate([htop, h3[1:-1], hbot], axis=0).astype(_CDT)

    o_ref[...] = conv3x3([hslab], T, wc2, wm2, s2, f2).astype(o_ref.dtype)


def _packed_conv_weights(w_torch, src_chans):
    Co = w_torch.shape[0]
    wc_parts, wm_parts = [], []
    off = 0
    for C in src_chans:
        w = w_torch[:, off:off + C]
        wt = jnp.transpose(w, (2, 1, 3, 0))
        z = jnp.zeros_like(wt[:, :, 0])
        dj0 = jnp.stack([wt[:, :, 1], wt[:, :, 0]], axis=2)
        dj1 = jnp.stack([wt[:, :, 2], wt[:, :, 1]], axis=2)
        wc = jnp.stack([dj0, dj1], axis=1)
        mdj0 = jnp.stack([z, wt[:, :, 2]], axis=2)
        mdj1 = jnp.stack([wt[:, :, 0], z], axis=2)
        wm = jnp.stack([mdj0, mdj1], axis=1)
        wc_parts.append(wc.reshape(3, 2 * C, 2 * Co))
        wm_parts.append(wm.reshape(3, 2 * C, 2 * Co))
        off += C
    wc = jnp.concatenate(wc_parts, axis=1).reshape(-1, 2 * Co)
    wm = jnp.concatenate(wm_parts, axis=1).reshape(-1, 2 * Co)
    return wc.astype(_CDT), wm.astype(_CDT)


def _dup(v):
    return jnp.concatenate([v, v]).astype(jnp.float32).reshape(1, -1)


def kernel(up_w, up_b, c1_w, c1_b, bn1_g, bn1_b, bn1_m, bn1_v,
           c2_w, c2_b, bn2_g, bn2_b, bn2_m, bn2_v, x1, x2):
    B, C1, H1, W1x = x1.shape
    _, C2, H2, W2 = x2.shape
    Cu = up_w.shape[1]
    Co1 = c1_w.shape[0]
    Co2 = c2_w.shape[0]
    W1 = W2 // 2

    x1h = jnp.transpose(x1, (0, 2, 3, 1)).astype(_CDT)
    x2p = jnp.transpose(x2, (0, 2, 3, 1)).reshape(B, H2, W1, 2 * C2).astype(_CDT)

    wt = jnp.transpose(up_w, (2, 3, 0, 1))
    wu0 = jnp.transpose(wt[0], (1, 0, 2)).reshape(C1, 2 * Cu).astype(_CDT)
    wu1 = jnp.transpose(wt[1], (1, 0, 2)).reshape(C1, 2 * Cu).astype(_CDT)
    bu = _dup(up_b)

    wc1, wm1 = _packed_conv_weights(c1_w, [C2, Cu])
    wc2, wm2 = _packed_conv_weights(c2_w, [Co1])
    K1, K2 = wc1.shape[0], wc2.shape[0]

    sc1 = bn1_g / jnp.sqrt(bn1_v + _EPS)
    sh1 = _dup((c1_b - bn1_m) * sc1 + bn1_b)
    sc1 = _dup(sc1)
    sc2 = bn2_g / jnp.sqrt(bn2_v + _EPS)
    sh2 = _dup((c2_b - bn2_m) * sc2 + bn2_b)
    sc2 = _dup(sc2)

    T = min(128, H2)
    T2 = T // 2
    grid = (B, H2 // T)

    in_specs = [
        pl.BlockSpec((None, T2, W1x, C1), lambda bi, r: (bi, r, 0, 0)),
        pl.BlockSpec((None, 1, W1x, C1),
                     lambda bi, r: (bi, jnp.maximum(r * T2 - 1, 0), 0, 0)),
        pl.BlockSpec((None, 1, W1x, C1),
                     lambda bi, r: (bi, jnp.minimum((r + 1) * T2, H1 - 1), 0, 0)),
        pl.BlockSpec((None, T, W1, 2 * C2), lambda bi, r: (bi, r, 0, 0)),
        pl.BlockSpec((None, 1, W1, 2 * C2),
                     lambda bi, r: (bi, jnp.maximum(r * T - 2, 0), 0, 0)),
        pl.BlockSpec((None, 1, W1, 2 * C2),
                     lambda bi, r: (bi, jnp.maximum(r * T - 1, 0), 0, 0)),
        pl.BlockSpec((None, 1, W1, 2 * C2),
                     lambda bi, r: (bi, jnp.minimum((r + 1) * T, H2 - 1), 0, 0)),
        pl.BlockSpec((None, 1, W1, 2 * C2),
                     lambda bi, r: (bi, jnp.minimum((r + 1) * T + 1, H2 - 1), 0, 0)),
        pl.BlockSpec((C1, 2 * Cu), lambda bi, r: (0, 0)),
        pl.BlockSpec((C1, 2 * Cu), lambda bi, r: (0, 0)),
        pl.BlockSpec((1, 2 * Cu), lambda bi, r: (0, 0)),
        pl.BlockSpec((K1, 2 * Co1), lambda bi, r: (0, 0)),
        pl.BlockSpec((K1, 2 * Co1), lambda bi, r: (0, 0)),
        pl.BlockSpec((1, 2 * Co1), lambda bi, r: (0, 0)),
        pl.BlockSpec((1, 2 * Co1), lambda bi, r: (0, 0)),
        pl.BlockSpec((K2, 2 * Co2), lambda bi, r: (0, 0)),
        pl.BlockSpec((K2, 2 * Co2), lambda bi, r: (0, 0)),
        pl.BlockSpec((1, 2 * Co2), lambda bi, r: (0, 0)),
        pl.BlockSpec((1, 2 * Co2), lambda bi, r: (0, 0)),
    ]

    flops = 2 * B * H2 * W2 * (C1 * Cu + 3 * 3 * (C2 + Cu) * Co1
                               + 3 * 3 * Co1 * Co2)
    bytes_acc = (B * H1 * W1x * C1 + B * H2 * W2 * C2 + B * H2 * W2 * Co2) * 2

    out = pl.pallas_call(
        _fused_body,
        out_shape=jax.ShapeDtypeStruct((B, H2, W1, 2 * Co2), _CDT),
        grid=grid,
        in_specs=in_specs,
        out_specs=pl.BlockSpec((None, T, W1, 2 * Co2),
                               lambda bi, r: (bi, r, 0, 0)),
        compiler_params=pltpu.CompilerParams(
            dimension_semantics=("parallel", "parallel"),
            vmem_limit_bytes=100 * 1024 * 1024),
        cost_estimate=pl.CostEstimate(flops=int(flops), transcendentals=0,
                                      bytes_accessed=int(bytes_acc)),
    )(x1h, x1h, x1h, x2p, x2p, x2p, x2p, x2p,
      wu0, wu1, bu, wc1, wm1, sc1, sh1, wc2, wm2, sc2, sh2)

    out = out.reshape(B, H2, W1, 2, Co2)
    return jnp.transpose(out, (0, 4, 1, 2, 3)).reshape(
        B, Co2, H2, W2).astype(jnp.float32)
